# baseline (speedup 1.0000x reference)
import jax
import jax.numpy as jnp
from jax.experimental import pallas as pl
from jax.experimental.pallas import tpu as pltpu

H = 256
BB = 256


def _dot_t(x, w):
    return jax.lax.dot_general(x, w, (((1,), (1,)), ((), ())),
                               preferred_element_type=jnp.float32)


def _dot(x, w):
    return jnp.dot(x, w, preferred_element_type=jnp.float32)


def _critic_body(st_ref, aug_ref, len_ref, wih_ref, bi_ref,
                 wh1_ref, bh1_ref, wh2_ref, bh2_ref,
                 a1_1_ref, h1_1_ref, b1_1_ref, w2_1_ref, b2_1_ref,
                 w3_1_ref, b3_1_ref, w4_1_ref, b4_1_ref, q_1_ref, qb_1_ref,
                 a1_2_ref, h1_2_ref, b1_2_ref, w2_2_ref, b2_2_ref,
                 w3_2_ref, b3_2_ref, w4_2_ref, b4_2_ref, q_2_ref, qb_2_ref,
                 o1_ref, o2_ref):
    T = st_ref.shape[0]
    bb = len_ref.shape[0]
    bf = jnp.bfloat16
    lenf = len_ref[...]
    wih = wih_ref[...]
    bi = bi_ref[...]
    wh1 = wh1_ref[...]
    wh2 = wh2_ref[...]
    bh1 = bh1_ref[...]
    bh2 = bh2_ref[...]
    lenb = jnp.broadcast_to(lenf, (bb, H))

    def gru_update(h, gi, wh, bh):
        gh = _dot(h.astype(bf), wh) + bh
        r = jax.nn.sigmoid(gi[:, :H] + gh[:, :H])
        z = jax.nn.sigmoid(gi[:, H:2 * H] + gh[:, H:2 * H])
        n = jnp.tanh(gi[:, 2 * H:] + r * gh[:, 2 * H:])
        return (1.0 - z) * n + z * h

    def step(t, carry):
        h1, h2 = carry
        s = st_ref[t]
        gi = _dot(s, wih) + bi
        u1 = gru_update(h1, gi[:, :3 * H], wh1, bh1)
        u2 = gru_update(h2, gi[:, 3 * H:], wh2, bh2)
        mk = lenb > t.astype(jnp.float32)
        h1 = jnp.where(mk, u1, h1)
        h2 = jnp.where(mk, u2, h2)
        return (h1, h2)

    def step4(i, carry):
        for j in range(4):
            carry = step(4 * i + j, carry)
        return carry

    trip = jnp.minimum(jnp.max(lenf), float(T)).astype(jnp.int32)
    h0 = jnp.zeros((bb, H), jnp.float32)
    h1, h2 = jax.lax.fori_loop(0, (trip + 3) // 4, step4, (h0, h0))

    aug = aug_ref[...]

    def mlp(h, a1_ref, h1_ref, b1_ref, w2_ref, b2_ref, w3_ref, b3_ref,
            w4_ref, b4_ref, q_ref, qb_ref):
        x = _dot_t(aug, a1_ref[...])
        x = x + _dot_t(h.astype(bf), h1_ref[...])
        x = jnp.maximum(x + b1_ref[...], 0.0)
        x = jnp.maximum(_dot_t(x.astype(bf), w2_ref[...]) + b2_ref[...], 0.0)
        x = jnp.maximum(_dot_t(x.astype(bf), w3_ref[...]) + b3_ref[...], 0.0)
        x = jnp.maximum(_dot_t(x.astype(bf), w4_ref[...]) + b4_ref[...], 0.0)
        return jnp.sum(x * q_ref[...], axis=1, keepdims=True) + qb_ref[...]

    o1_ref[...] = mlp(h1, a1_1_ref, h1_1_ref, b1_1_ref, w2_1_ref, b2_1_ref,
                      w3_1_ref, b3_1_ref, w4_1_ref, b4_1_ref, q_1_ref, qb_1_ref)
    o2_ref[...] = mlp(h2, a1_2_ref, h1_2_ref, b1_2_ref, w2_2_ref, b2_2_ref,
                      w3_2_ref, b3_2_ref, w4_2_ref, b4_2_ref, q_2_ref, qb_2_ref)


@jax.jit
def kernel(state, action, lengths,
           g1_Wih, g1_Whh, g1_bih, g1_bhh,
           fc1_1_w, fc1_1_b, fc2_1_w, fc2_1_b, fc3_1_w, fc3_1_b,
           fc4_1_w, fc4_1_b, q_1_w, q_1_b,
           g2_Wih, g2_Whh, g2_bih, g2_bhh,
           fc1_2_w, fc1_2_b, fc2_2_w, fc2_2_b, fc3_2_w, fc3_2_b,
           fc4_2_w, fc4_2_b, q_2_w, q_2_b):
    B, T, D = state.shape
    A = action.shape[1]
    bf = jnp.bfloat16

    bb = min(BB, B)
    G = B // bb
    order = []
    for k in range(0, G // 2, 2):
        order += [G - 1 - k, k]
    for k in range(1, G // 2, 2):
        order += [G - 1 - k, k]
    if G % 2:
        order.append(G // 2)
    iorder = [0] * G
    for j, b in enumerate(order):
        iorder[b] = j
    lb = jnp.clip(lengths, 1, T) - 1
    oh = (lb[:, None] == jnp.arange(T)[None, :]).astype(jnp.int32)
    ex = jnp.cumsum(oh, axis=0)
    rank = jnp.take_along_axis(ex, lb[:, None], 1)[:, 0] - 1
    tot = ex[-1]
    off = jnp.concatenate([jnp.zeros((1,), jnp.int32),
                           jnp.cumsum(tot)[:-1]])
    pos = off[lb] + rank
    inv = jnp.array(iorder, jnp.int32)[pos // bb] * bb + pos % bb
    perm = jnp.zeros((B,), jnp.int32).at[inv].set(
        jnp.arange(B, dtype=jnp.int32))
    state_p = state.astype(bf)[perm]
    lengths = lengths[perm]

    st = jnp.transpose(state_p, (1, 0, 2))
    aug = jnp.concatenate([state_p[:, 0, :],
                           action.astype(bf)[perm]], -1)
    lenf = lengths.astype(jnp.float32)[:, None]

    wih = jnp.concatenate([g1_Wih, g2_Wih], 0).T.astype(bf)
    bi = jnp.concatenate([g1_bih, g2_bih])[None]
    wh1 = g1_Whh.T.astype(bf)
    wh2 = g2_Whh.T.astype(bf)
    bh1 = g1_bhh[None]
    bh2 = g2_bhh[None]

    na = D + A

    def prep_mlp(w1, b1, w2, b2, w3, b3, w4, b4, qw, qb):
        return (w1[:, :na].astype(bf), w1[:, na:].astype(bf), b1[None],
                w2.astype(bf), b2[None], w3.astype(bf), b3[None],
                w4.astype(bf), b4[None], qw, qb[None])

    m1 = prep_mlp(fc1_1_w, fc1_1_b, fc2_1_w, fc2_1_b, fc3_1_w, fc3_1_b,
                  fc4_1_w, fc4_1_b, q_1_w, q_1_b)
    m2 = prep_mlp(fc1_2_w, fc1_2_b, fc2_2_w, fc2_2_b, fc3_2_w, fc3_2_b,
                  fc4_2_w, fc4_2_b, q_2_w, q_2_b)

    inputs = (st, aug, lenf, wih, bi, wh1, bh1, wh2, bh2) + m1 + m2

    def wspec(x):
        return pl.BlockSpec(x.shape, lambda i: (0,) * x.ndim)

    in_specs = [
        pl.BlockSpec((T, bb, D), lambda i: (0, i, 0)),
        pl.BlockSpec((bb, na), lambda i: (i, 0)),
        pl.BlockSpec((bb, 1), lambda i: (i, 0)),
    ] + [wspec(x) for x in inputs[3:]]

    out1, out2 = pl.pallas_call(
        _critic_body,
        grid=(B // bb,),
        in_specs=in_specs,
        out_specs=[pl.BlockSpec((bb, 1), lambda i: (i, 0))] * 2,
        out_shape=[jax.ShapeDtypeStruct((B, 1), jnp.float32)] * 2,
        compiler_params=pltpu.CompilerParams(
            dimension_semantics=("parallel",),
            vmem_limit_bytes=56 * 1024 * 1024,
        ),
    )(*inputs)
    return (out1[inv], out2[inv])


# speedup vs baseline: 1.1687x; 1.1687x over previous
import jax
import jax.numpy as jnp
from jax.experimental import pallas as pl
from jax.experimental.pallas import tpu as pltpu

H = 256
BB = 256


def _dot_t(x, w):
    return jax.lax.dot_general(x, w, (((1,), (1,)), ((), ())),
                               preferred_element_type=jnp.float32)


def _dot(x, w):
    return jnp.dot(x, w, preferred_element_type=jnp.float32)


def _critic_body(st_ref, aug_ref, len_ref, wih_ref, bi_ref,
                 wh1_ref, bh1_ref, wh2_ref, bh2_ref,
                 a1_1_ref, h1_1_ref, b1_1_ref, w2_1_ref, b2_1_ref,
                 w3_1_ref, b3_1_ref, w4_1_ref, b4_1_ref, q_1_ref, qb_1_ref,
                 a1_2_ref, h1_2_ref, b1_2_ref, w2_2_ref, b2_2_ref,
                 w3_2_ref, b3_2_ref, w4_2_ref, b4_2_ref, q_2_ref, qb_2_ref,
                 o1_ref, o2_ref):
    T = st_ref.shape[0]
    bb = len_ref.shape[0]
    bf = jnp.bfloat16
    lenf = len_ref[...]
    wih = wih_ref[...]
    bi = bi_ref[...]
    wh1 = wh1_ref[...]
    wh2 = wh2_ref[...]
    bh1 = bh1_ref[...]
    bh2 = bh2_ref[...]
    lenb = jnp.broadcast_to(lenf, (bb, H))

    def gru_update(h, gi, wh, bh):
        gh = _dot(h.astype(bf), wh) + bh
        r = jax.nn.sigmoid(gi[:, :H] + gh[:, :H])
        z = jax.nn.sigmoid(gi[:, H:2 * H] + gh[:, H:2 * H])
        n = jnp.tanh(gi[:, 2 * H:] + r * gh[:, 2 * H:])
        return (1.0 - z) * n + z * h

    def step(t, carry):
        h1, h2 = carry
        s = st_ref[t]
        gi = _dot(s, wih) + bi
        u1 = gru_update(h1, gi[:, :3 * H], wh1, bh1)
        u2 = gru_update(h2, gi[:, 3 * H:], wh2, bh2)
        mk = lenb > t.astype(jnp.float32)
        h1 = jnp.where(mk, u1, h1)
        h2 = jnp.where(mk, u2, h2)
        return (h1, h2)

    def step4(i, carry):
        for j in range(4):
            carry = step(4 * i + j, carry)
        return carry

    trip = jnp.minimum(jnp.max(lenf), float(T)).astype(jnp.int32)
    h0 = jnp.zeros((bb, H), jnp.float32)
    h1, h2 = jax.lax.fori_loop(0, (trip + 3) // 4, step4, (h0, h0))

    aug = aug_ref[...]

    def mlp(h, a1_ref, h1_ref, b1_ref, w2_ref, b2_ref, w3_ref, b3_ref,
            w4_ref, b4_ref, q_ref, qb_ref):
        x = _dot_t(aug, a1_ref[...])
        x = x + _dot_t(h.astype(bf), h1_ref[...])
        x = jnp.maximum(x + b1_ref[...], 0.0)
        x = jnp.maximum(_dot_t(x.astype(bf), w2_ref[...]) + b2_ref[...], 0.0)
        x = jnp.maximum(_dot_t(x.astype(bf), w3_ref[...]) + b3_ref[...], 0.0)
        x = jnp.maximum(_dot_t(x.astype(bf), w4_ref[...]) + b4_ref[...], 0.0)
        return jnp.sum(x * q_ref[...], axis=1, keepdims=True) + qb_ref[...]

    o1_ref[...] = mlp(h1, a1_1_ref, h1_1_ref, b1_1_ref, w2_1_ref, b2_1_ref,
                      w3_1_ref, b3_1_ref, w4_1_ref, b4_1_ref, q_1_ref, qb_1_ref)
    o2_ref[...] = mlp(h2, a1_2_ref, h1_2_ref, b1_2_ref, w2_2_ref, b2_2_ref,
                      w3_2_ref, b3_2_ref, w4_2_ref, b4_2_ref, q_2_ref, qb_2_ref)


@jax.jit
def kernel(state, action, lengths,
           g1_Wih, g1_Whh, g1_bih, g1_bhh,
           fc1_1_w, fc1_1_b, fc2_1_w, fc2_1_b, fc3_1_w, fc3_1_b,
           fc4_1_w, fc4_1_b, q_1_w, q_1_b,
           g2_Wih, g2_Whh, g2_bih, g2_bhh,
           fc1_2_w, fc1_2_b, fc2_2_w, fc2_2_b, fc3_2_w, fc3_2_b,
           fc4_2_w, fc4_2_b, q_2_w, q_2_b):
    B, T, D = state.shape
    A = action.shape[1]
    bf = jnp.bfloat16

    bb = min(BB, B)
    perm = jnp.argsort(lengths)
    inv = jnp.zeros((B,), jnp.int32).at[perm].set(
        jnp.arange(B, dtype=jnp.int32))
    state_p = state.astype(bf)[perm]
    lengths = lengths[perm]

    st = jnp.transpose(state_p, (1, 0, 2))
    aug = jnp.concatenate([state_p[:, 0, :],
                           action.astype(bf)[perm]], -1)
    lenf = lengths.astype(jnp.float32)[:, None]

    wih = jnp.concatenate([g1_Wih, g2_Wih], 0).T.astype(bf)
    bi = jnp.concatenate([g1_bih, g2_bih])[None]
    wh1 = g1_Whh.T.astype(bf)
    wh2 = g2_Whh.T.astype(bf)
    bh1 = g1_bhh[None]
    bh2 = g2_bhh[None]

    na = D + A

    def prep_mlp(w1, b1, w2, b2, w3, b3, w4, b4, qw, qb):
        return (w1[:, :na].astype(bf), w1[:, na:].astype(bf), b1[None],
                w2.astype(bf), b2[None], w3.astype(bf), b3[None],
                w4.astype(bf), b4[None], qw, qb[None])

    m1 = prep_mlp(fc1_1_w, fc1_1_b, fc2_1_w, fc2_1_b, fc3_1_w, fc3_1_b,
                  fc4_1_w, fc4_1_b, q_1_w, q_1_b)
    m2 = prep_mlp(fc1_2_w, fc1_2_b, fc2_2_w, fc2_2_b, fc3_2_w, fc3_2_b,
                  fc4_2_w, fc4_2_b, q_2_w, q_2_b)

    inputs = (st, aug, lenf, wih, bi, wh1, bh1, wh2, bh2) + m1 + m2

    def wspec(x):
        return pl.BlockSpec(x.shape, lambda i: (0,) * x.ndim)

    in_specs = [
        pl.BlockSpec((T, bb, D), lambda i: (0, i, 0)),
        pl.BlockSpec((bb, na), lambda i: (i, 0)),
        pl.BlockSpec((bb, 1), lambda i: (i, 0)),
    ] + [wspec(x) for x in inputs[3:]]

    out1, out2 = pl.pallas_call(
        _critic_body,
        grid=(B // bb,),
        in_specs=in_specs,
        out_specs=[pl.BlockSpec((bb, 1), lambda i: (i, 0))] * 2,
        out_shape=[jax.ShapeDtypeStruct((B, 1), jnp.float32)] * 2,
        compiler_params=pltpu.CompilerParams(
            dimension_semantics=("arbitrary",),
            vmem_limit_bytes=56 * 1024 * 1024,
        ),
    )(*inputs)
    return (out1[inv], out2[inv])
